# baseline (speedup 1.0000x reference)
import functools

import jax
import jax.numpy as jnp
from jax import lax
from jax.experimental import pallas as pl
from jax.experimental.pallas import tpu as pltpu
from jax.experimental.pallas import tpu_sc as plsc

N_NODES = 10000
N_EDGES = 320000
IN_CH = 128
NREGION = 10

_NC, _NS = 2, 16
_NW = _NC * _NS
CH = 80
NCH = N_EDGES // (_NW * CH)
DEG_PAD = 10240


def _sc_mesh():
    return plsc.VectorSubcoreMesh(
        core_axis_name="c", subcore_axis_name="s",
        num_cores=_NC, num_subcores=_NS)


@functools.partial(
    pl.kernel,
    out_type=jax.ShapeDtypeStruct((_NC * DEG_PAD,), jnp.float32),
    mesh=_sc_mesh(),
    scratch_types=[
        pltpu.VMEM((NCH, CH), jnp.int32),
        pltpu.VMEM((CH,), jnp.float32),
        pltpu.VMEM((640,), jnp.float32),
        pltpu.VMEM_SHARED((DEG_PAD,), jnp.float32),
    ],
)
def _deg_kernel(dst_hbm, degp_hbm, idx_v, ones_v, zeros_v, acc_sh):
    c = lax.axis_index("c")
    s = lax.axis_index("s")
    w = c * _NS + s
    for i in range(CH // 16):
        ones_v[pl.ds(i * 16, 16)] = jnp.ones((16,), jnp.float32)
    for i in range(640 // 16):
        zeros_v[pl.ds(i * 16, 16)] = jnp.zeros((16,), jnp.float32)
    pltpu.sync_copy(zeros_v, acc_sh.at[pl.ds(s * 640, 640)])
    pltpu.sync_copy(dst_hbm.at[w], idx_v)
    plsc.subcore_barrier()

    def body(j, carry):
        pltpu.sync_copy(ones_v, acc_sh.at[idx_v.at[j]], add=True)
        return carry

    lax.fori_loop(0, NCH, body, 0)
    plsc.subcore_barrier()
    pltpu.sync_copy(acc_sh.at[pl.ds(s * 640, 640)],
                    degp_hbm.at[pl.ds(c * DEG_PAD + s * 640, 640)])


_HCH = IN_CH // 2
_NPAD = 10240
_RPT = _NPAD // _NS
_CH2 = 128
_NCH2 = 158
_EPTP = _NCH2 * _CH2


@functools.partial(
    pl.kernel,
    out_type=jax.ShapeDtypeStruct((_NC * _NPAD, _HCH), jnp.float32),
    mesh=_sc_mesh(),
    scratch_types=[
        pltpu.VMEM((_NCH2, _CH2), jnp.int32),
        pltpu.VMEM((_NCH2, _CH2), jnp.int32),
        pltpu.VMEM((2, _CH2, _HCH), jnp.float32),
        pltpu.VMEM((128, _HCH), jnp.float32),
        pltpu.VMEM_SHARED((_NPAD, _HCH), jnp.float32),
        pltpu.SemaphoreType.DMA,
        pltpu.SemaphoreType.DMA,
        pltpu.SemaphoreType.DMA,
        pltpu.SemaphoreType.DMA,
    ],
    compiler_params=pltpu.CompilerParams(use_tc_tiling_on_sc=False),
)
def _msg_kernel(yw_hbm, src_hbm, dst_hbm, sp_hbm,
                sidx_v, didx_v, rows_v, zrow_v, acc_sh,
                sg0, sg1, ss0, ss1):
    c = lax.axis_index("c")
    s = lax.axis_index("s")

    def zbody(j, carry):
        for i in range(_HCH // 16):
            zrow_v[j, pl.ds(i * 16, 16)] = jnp.zeros((16,), jnp.float32)
        return carry

    lax.fori_loop(0, 128, zbody, 0)
    for k in range(_RPT // 128):
        pltpu.sync_copy(zrow_v, acc_sh.at[pl.ds(s * _RPT + k * 128, 128)])
    pltpu.sync_copy(src_hbm.at[s], sidx_v)
    pltpu.sync_copy(dst_hbm.at[s], didx_v)

    coff = jnp.full((16,), 0, jnp.int32) + c

    def ibody(j, carry):
        for i in range(_CH2 // 16):
            v = sidx_v[j, pl.ds(i * 16, 16)]
            sidx_v[j, pl.ds(i * 16, 16)] = v * 2 + coff
        return carry

    lax.fori_loop(0, _NCH2, ibody, 0)
    plsc.subcore_barrier()

    nit = _NCH2 // 2
    pltpu.async_copy(yw_hbm.at[sidx_v.at[0]], rows_v.at[0], sg0)
    pltpu.async_copy(yw_hbm.at[sidx_v.at[1]], rows_v.at[1], sg1)

    def body(t, carry):
        j0 = 2 * t
        pltpu.make_async_copy(
            yw_hbm.at[sidx_v.at[j0]], rows_v.at[0], sg0).wait()
        pltpu.async_copy(rows_v.at[0], acc_sh.at[didx_v.at[j0]], ss0,
                         add=True)

        @pl.when(t >= 1)
        def _():
            pltpu.make_async_copy(
                rows_v.at[0], acc_sh.at[didx_v.at[j0]], ss0).wait()

        @pl.when(t < nit - 1)
        def _():
            pltpu.async_copy(yw_hbm.at[sidx_v.at[j0 + 2]], rows_v.at[0], sg0)

        pltpu.make_async_copy(
            yw_hbm.at[sidx_v.at[j0 + 1]], rows_v.at[1], sg1).wait()
        pltpu.async_copy(rows_v.at[1], acc_sh.at[didx_v.at[j0 + 1]], ss1,
                         add=True)

        @pl.when(t >= 1)
        def _():
            pltpu.make_async_copy(
                rows_v.at[1], acc_sh.at[didx_v.at[j0 + 1]], ss1).wait()

        @pl.when(t < nit - 1)
        def _():
            pltpu.async_copy(yw_hbm.at[sidx_v.at[j0 + 3]], rows_v.at[1], sg1)

        return carry

    lax.fori_loop(0, nit, body, 0)
    pltpu.make_async_copy(
        rows_v.at[0], acc_sh.at[didx_v.at[_NCH2 - 2]], ss0).wait()
    pltpu.make_async_copy(
        rows_v.at[1], acc_sh.at[didx_v.at[_NCH2 - 1]], ss1).wait()
    plsc.subcore_barrier()
    for k in range(_RPT // 128):
        pltpu.sync_copy(
            acc_sh.at[pl.ds(s * _RPT + k * 128, 128)],
            sp_hbm.at[pl.ds(c * _NPAD + s * _RPT + k * 128, 128)])


_RB = 400


def _xw_body(state_ref, wc_ref, xw_ref):
    xw_ref[...] = jnp.dot(state_ref[...], wc_ref[...],
                          preferred_element_type=jnp.float32)


def _yw_body(xw_ref, d0_ref, d1_ref, yw_ref, dinv_ref):
    deg = d0_ref[...] + d1_ref[...] + 1.0
    dinv = lax.rsqrt(deg)
    yw_ref[...] = xw_ref[...] * dinv
    dinv_ref[...] = dinv


def _xsxd_body(s0_ref, s1_ref, yw_ref, dinv_ref, state_ref, bc_ref,
               w1a_ref, w1b_ref, b1_ref, xs_ref, xd_ref):
    s_full = jnp.concatenate([s0_ref[...], s1_ref[...]], axis=-1)
    gcn = dinv_ref[...] * (s_full + yw_ref[...]) + bc_ref[...]
    x = jnp.maximum(gcn, 0.0) + state_ref[...]
    xs_ref[...] = jnp.dot(x, w1a_ref[...],
                          preferred_element_type=jnp.float32) + b1_ref[...]
    xd_ref[...] = jnp.dot(x, w1b_ref[...],
                          preferred_element_type=jnp.float32)


_GB = 40


def _act_body(xs_ref, xd_ref, w20_ref, w21_ref, b2_ref, c0_ref, c1_ref):
    xd = xd_ref[...]
    w20 = w20_ref[...][None]
    w21 = w21_ref[...][None]
    b20 = b2_ref[0:1, 0:1]
    b21 = b2_ref[0:1, 1:2]
    for i in range(NREGION):
        z = xs_ref[:, i:i + 1, :] + xd
        h = jnp.where(z >= 0, z, 0.01 * z)
        c0_ref[:, i * 10:(i + 1) * 10] = jnp.sum(h * w20, axis=-1) + b20
        c1_ref[:, i * 10:(i + 1) * 10] = jnp.sum(h * w21, axis=-1) + b21


def _ratio_body(c0_ref, c1_ref, out_ref):
    def softplus(v):
        return jnp.maximum(v, 0.0) + jnp.log1p(jnp.exp(-jnp.abs(v)))

    p0 = softplus(c0_ref[...])
    p1 = softplus(c1_ref[...])
    a = p0 / (p0 + p1 + 1e-10)
    out_ref[...] = jnp.where(a < 0, 0.0, a)


def kernel(state, edge_index, edges, Wc, bc, W1, b1, W2, b2):
    del edges
    src2 = edge_index[0].reshape(_NW, NCH, CH)
    dst2 = edge_index[1].reshape(_NW, NCH, CH)

    nb = N_NODES // _RB
    xw = pl.pallas_call(
        _xw_body,
        grid=(nb,),
        in_specs=[
            pl.BlockSpec((_RB, IN_CH), lambda b: (b, 0)),
            pl.BlockSpec((IN_CH, IN_CH), lambda b: (0, 0)),
        ],
        out_specs=pl.BlockSpec((_RB, IN_CH), lambda b: (b, 0)),
        out_shape=jax.ShapeDtypeStruct((N_NODES, IN_CH), jnp.float32),
    )(state, Wc)

    degp = _deg_kernel(dst2)
    degp = degp.reshape(_NC, DEG_PAD)
    d0 = degp[0, :N_NODES].reshape(N_NODES, 1)
    d1 = degp[1, :N_NODES].reshape(N_NODES, 1)

    yw, dinv = pl.pallas_call(
        _yw_body,
        grid=(nb,),
        in_specs=[
            pl.BlockSpec((_RB, IN_CH), lambda b: (b, 0)),
            pl.BlockSpec((_RB, 1), lambda b: (b, 0)),
            pl.BlockSpec((_RB, 1), lambda b: (b, 0)),
        ],
        out_specs=[
            pl.BlockSpec((_RB, IN_CH), lambda b: (b, 0)),
            pl.BlockSpec((_RB, 1), lambda b: (b, 0)),
        ],
        out_shape=[
            jax.ShapeDtypeStruct((N_NODES, IN_CH), jnp.float32),
            jax.ShapeDtypeStruct((N_NODES, 1), jnp.float32),
        ],
    )(xw, d0, d1)

    npad_e = _EPTP - N_EDGES // _NS
    pad_src = jnp.zeros((_NS, npad_e), jnp.int32)
    pad_dst = (N_NODES
               + (jnp.arange(_NS * npad_e, dtype=jnp.int32) % (_NPAD - N_NODES))
               ).reshape(_NS, npad_e)
    srct = jnp.concatenate(
        [edge_index[0].reshape(_NS, -1), pad_src], axis=1
    ).reshape(_NS, _NCH2, _CH2)
    dstt = jnp.concatenate(
        [edge_index[1].reshape(_NS, -1), pad_dst], axis=1
    ).reshape(_NS, _NCH2, _CH2)
    sp = _msg_kernel(yw.reshape(2 * N_NODES, _HCH), srct, dstt)
    s0 = sp[:N_NODES]
    s1 = sp[_NPAD:_NPAD + N_NODES]

    w1a = W1[:, :IN_CH].T
    w1b = W1[:, IN_CH:].T
    xs, xd = pl.pallas_call(
        _xsxd_body,
        grid=(nb,),
        in_specs=[
            pl.BlockSpec((_RB, _HCH), lambda b: (b, 0)),
            pl.BlockSpec((_RB, _HCH), lambda b: (b, 0)),
            pl.BlockSpec((_RB, IN_CH), lambda b: (b, 0)),
            pl.BlockSpec((_RB, 1), lambda b: (b, 0)),
            pl.BlockSpec((_RB, IN_CH), lambda b: (b, 0)),
            pl.BlockSpec((1, IN_CH), lambda b: (0, 0)),
            pl.BlockSpec((IN_CH, IN_CH), lambda b: (0, 0)),
            pl.BlockSpec((IN_CH, IN_CH), lambda b: (0, 0)),
            pl.BlockSpec((1, IN_CH), lambda b: (0, 0)),
        ],
        out_specs=[
            pl.BlockSpec((_RB, IN_CH), lambda b: (b, 0)),
            pl.BlockSpec((_RB, IN_CH), lambda b: (b, 0)),
        ],
        out_shape=[
            jax.ShapeDtypeStruct((N_NODES, IN_CH), jnp.float32),
            jax.ShapeDtypeStruct((N_NODES, IN_CH), jnp.float32),
        ],
    )(s0, s1, yw, dinv, state, bc.reshape(1, IN_CH), w1a, w1b,
      b1.reshape(1, IN_CH))

    ngraph = N_NODES // NREGION
    npair = NREGION * NREGION
    xs3 = xs.reshape(ngraph, NREGION, IN_CH)
    xd3 = xd.reshape(ngraph, NREGION, IN_CH)
    c0, c1 = pl.pallas_call(
        _act_body,
        grid=(ngraph // _GB,),
        in_specs=[
            pl.BlockSpec((_GB, NREGION, IN_CH), lambda b: (b, 0, 0)),
            pl.BlockSpec((_GB, NREGION, IN_CH), lambda b: (b, 0, 0)),
            pl.BlockSpec((1, IN_CH), lambda b: (0, 0)),
            pl.BlockSpec((1, IN_CH), lambda b: (0, 0)),
            pl.BlockSpec((1, 2), lambda b: (0, 0)),
        ],
        out_specs=[
            pl.BlockSpec((_GB, npair), lambda b: (b, 0)),
            pl.BlockSpec((_GB, npair), lambda b: (b, 0)),
        ],
        out_shape=[
            jax.ShapeDtypeStruct((ngraph, npair), jnp.float32),
            jax.ShapeDtypeStruct((ngraph, npair), jnp.float32),
        ],
    )(xs3, xd3, W2[0:1, :], W2[1:2, :], b2.reshape(1, 2))

    action = pl.pallas_call(
        _ratio_body,
        grid=(5,),
        in_specs=[
            pl.BlockSpec((ngraph // 5, npair), lambda b: (b, 0)),
            pl.BlockSpec((ngraph // 5, npair), lambda b: (b, 0)),
        ],
        out_specs=pl.BlockSpec((ngraph // 5, npair), lambda b: (b, 0)),
        out_shape=jax.ShapeDtypeStruct((ngraph, npair), jnp.float32),
    )(c0, c1)
    return action.reshape(ngraph, NREGION, NREGION)


# speedup vs baseline: 1.4087x; 1.4087x over previous
import functools

import jax
import jax.numpy as jnp
from jax import lax
from jax.experimental import pallas as pl
from jax.experimental.pallas import tpu as pltpu
from jax.experimental.pallas import tpu_sc as plsc

N_NODES = 10000
N_EDGES = 320000
IN_CH = 128
NREGION = 10

_NC, _NS = 2, 16
_NW = _NC * _NS
CH = 80
NCH = N_EDGES // (_NW * CH)
DEG_PAD = 10240


def _sc_mesh():
    return plsc.VectorSubcoreMesh(
        core_axis_name="c", subcore_axis_name="s",
        num_cores=_NC, num_subcores=_NS)


@functools.partial(
    pl.kernel,
    out_type=jax.ShapeDtypeStruct((_NC * DEG_PAD,), jnp.float32),
    mesh=_sc_mesh(),
    scratch_types=[
        pltpu.VMEM((NCH, CH), jnp.int32),
        pltpu.VMEM((CH,), jnp.float32),
        pltpu.VMEM((640,), jnp.float32),
        pltpu.VMEM_SHARED((DEG_PAD,), jnp.float32),
    ],
)
def _deg_kernel(dst_hbm, degp_hbm, idx_v, ones_v, zeros_v, acc_sh):
    c = lax.axis_index("c")
    s = lax.axis_index("s")
    w = c * _NS + s
    for i in range(CH // 16):
        ones_v[pl.ds(i * 16, 16)] = jnp.ones((16,), jnp.float32)
    for i in range(640 // 16):
        zeros_v[pl.ds(i * 16, 16)] = jnp.zeros((16,), jnp.float32)
    pltpu.sync_copy(zeros_v, acc_sh.at[pl.ds(s * 640, 640)])
    pltpu.sync_copy(dst_hbm.at[w], idx_v)
    plsc.subcore_barrier()

    def body(j, carry):
        pltpu.sync_copy(ones_v, acc_sh.at[idx_v.at[j]], add=True)
        return carry

    lax.fori_loop(0, NCH, body, 0)
    plsc.subcore_barrier()
    pltpu.sync_copy(acc_sh.at[pl.ds(s * 640, 640)],
                    degp_hbm.at[pl.ds(c * DEG_PAD + s * 640, 640)])


_HCH = IN_CH // 2
_NPAD = 12800
_RPT = _NPAD // _NS
_CH2 = 80
_NCH2 = 250
_EPTP = _NCH2 * _CH2


@functools.partial(
    pl.kernel,
    out_type=jax.ShapeDtypeStruct((_NC * _NPAD, _HCH), jnp.float32),
    mesh=_sc_mesh(),
    scratch_types=[
        pltpu.VMEM((_NCH2, _CH2), jnp.int32),
        pltpu.VMEM((_NCH2, _CH2), jnp.int32),
        pltpu.VMEM((2, _CH2, _HCH), jnp.float32),
        pltpu.VMEM((160, _HCH), jnp.float32),
        pltpu.VMEM_SHARED((_NPAD, _HCH), jnp.float32),
        pltpu.SemaphoreType.DMA,
        pltpu.SemaphoreType.DMA,
        pltpu.SemaphoreType.DMA,
        pltpu.SemaphoreType.DMA,
    ],
    compiler_params=pltpu.CompilerParams(use_tc_tiling_on_sc=False),
)
def _msg_kernel(yw_hbm, src_hbm, dst_hbm, sp_hbm,
                sidx_v, didx_v, rows_v, zrow_v, acc_sh,
                sg0, sg1, ss0, ss1):
    c = lax.axis_index("c")
    s = lax.axis_index("s")

    def zbody(j, carry):
        for i in range(_HCH // 16):
            zrow_v[j, pl.ds(i * 16, 16)] = jnp.zeros((16,), jnp.float32)
        return carry

    lax.fori_loop(0, 160, zbody, 0)
    for k in range(_RPT // 160):
        pltpu.sync_copy(zrow_v, acc_sh.at[pl.ds(s * _RPT + k * 160, 160)])
    pltpu.sync_copy(src_hbm.at[s], sidx_v)
    pltpu.sync_copy(dst_hbm.at[s], didx_v)

    coff = jnp.full((16,), 0, jnp.int32) + c

    def ibody(j, carry):
        for i in range(_CH2 // 16):
            v = sidx_v[j, pl.ds(i * 16, 16)]
            sidx_v[j, pl.ds(i * 16, 16)] = v * 2 + coff
        return carry

    lax.fori_loop(0, _NCH2, ibody, 0)
    plsc.subcore_barrier()

    nit = _NCH2 // 2
    pltpu.async_copy(yw_hbm.at[sidx_v.at[0]], rows_v.at[0], sg0)
    pltpu.async_copy(yw_hbm.at[sidx_v.at[1]], rows_v.at[1], sg1)

    def body(t, carry):
        j0 = 2 * t
        pltpu.make_async_copy(
            yw_hbm.at[sidx_v.at[j0]], rows_v.at[0], sg0).wait()
        pltpu.async_copy(rows_v.at[0], acc_sh.at[didx_v.at[j0]], ss0,
                         add=True)

        @pl.when(t >= 1)
        def _():
            pltpu.make_async_copy(
                rows_v.at[0], acc_sh.at[didx_v.at[j0]], ss0).wait()

        @pl.when(t < nit - 1)
        def _():
            pltpu.async_copy(yw_hbm.at[sidx_v.at[j0 + 2]], rows_v.at[0], sg0)

        pltpu.make_async_copy(
            yw_hbm.at[sidx_v.at[j0 + 1]], rows_v.at[1], sg1).wait()
        pltpu.async_copy(rows_v.at[1], acc_sh.at[didx_v.at[j0 + 1]], ss1,
                         add=True)

        @pl.when(t >= 1)
        def _():
            pltpu.make_async_copy(
                rows_v.at[1], acc_sh.at[didx_v.at[j0 + 1]], ss1).wait()

        @pl.when(t < nit - 1)
        def _():
            pltpu.async_copy(yw_hbm.at[sidx_v.at[j0 + 3]], rows_v.at[1], sg1)

        return carry

    lax.fori_loop(0, nit, body, 0)
    pltpu.make_async_copy(
        rows_v.at[0], acc_sh.at[didx_v.at[_NCH2 - 2]], ss0).wait()
    pltpu.make_async_copy(
        rows_v.at[1], acc_sh.at[didx_v.at[_NCH2 - 1]], ss1).wait()
    plsc.subcore_barrier()
    for k in range(_RPT // 160):
        pltpu.sync_copy(
            acc_sh.at[pl.ds(s * _RPT + k * 160, 160)],
            sp_hbm.at[pl.ds(c * _NPAD + s * _RPT + k * 160, 160)])


_RB = 400


def _xw_body(state_ref, wc_ref, xw_ref):
    xw_ref[...] = jnp.dot(state_ref[...], wc_ref[...],
                          preferred_element_type=jnp.float32)


def _dexp_body(d0_ref, d1_ref, dinvb_ref):
    dinv = lax.rsqrt(d0_ref[...] + d1_ref[...] + 1.0)
    ones = jnp.ones((1, IN_CH), jnp.float32)
    for r in range(8):
        dinvb_ref[r * 128:(r + 1) * 128, :] = lax.dot_general(
            dinv[r:r + 1, :], ones, (((0,), (0,)), ((), ())),
            preferred_element_type=jnp.float32)


def _yw_body(xw_ref, dv_ref, yw_ref):
    yw_ref[...] = xw_ref[...] * dv_ref[...]


def _xsxd_body(s0_ref, s1_ref, yw_ref, dv_ref, state_ref, bc_ref,
               w1a_ref, w1b_ref, b1_ref, xs_ref, xd_ref):
    s_full = jnp.concatenate([s0_ref[...], s1_ref[...]], axis=-1)
    gcn = dv_ref[...] * (s_full + yw_ref[...]) + bc_ref[...]
    x = jnp.maximum(gcn, 0.0) + state_ref[...]
    xs_ref[...] = jnp.dot(x, w1a_ref[...],
                          preferred_element_type=jnp.float32) + b1_ref[...]
    xd_ref[...] = jnp.dot(x, w1b_ref[...],
                          preferred_element_type=jnp.float32)


_GB = 40


def _act_body(xs_ref, xd_ref, w20_ref, w21_ref, b2_ref, c0_ref, c1_ref):
    xd = xd_ref[...]
    w20 = w20_ref[...][None]
    w21 = w21_ref[...][None]
    b20 = b2_ref[0:1, 0:1]
    b21 = b2_ref[0:1, 1:2]
    for i in range(NREGION):
        z = xs_ref[:, i:i + 1, :] + xd
        h = jnp.where(z >= 0, z, 0.01 * z)
        c0_ref[:, i * 10:(i + 1) * 10] = jnp.sum(h * w20, axis=-1) + b20
        c1_ref[:, i * 10:(i + 1) * 10] = jnp.sum(h * w21, axis=-1) + b21


def _ratio_body(c0_ref, c1_ref, out_ref):
    def softplus(v):
        return jnp.maximum(v, 0.0) + jnp.log1p(jnp.exp(-jnp.abs(v)))

    p0 = softplus(c0_ref[...])
    p1 = softplus(c1_ref[...])
    a = p0 / (p0 + p1 + 1e-10)
    out_ref[...] = jnp.where(a < 0, 0.0, a)


def kernel(state, edge_index, edges, Wc, bc, W1, b1, W2, b2):
    del edges
    src2 = edge_index[0].reshape(_NW, NCH, CH)
    dst2 = edge_index[1].reshape(_NW, NCH, CH)

    nb = N_NODES // _RB
    xw = pl.pallas_call(
        _xw_body,
        grid=(nb,),
        in_specs=[
            pl.BlockSpec((_RB, IN_CH), lambda b: (b, 0)),
            pl.BlockSpec((IN_CH, IN_CH), lambda b: (0, 0)),
        ],
        out_specs=pl.BlockSpec((_RB, IN_CH), lambda b: (b, 0)),
        out_shape=jax.ShapeDtypeStruct((N_NODES, IN_CH), jnp.float32),
    )(state, Wc)

    degp = _deg_kernel(dst2)
    d0r = degp[:DEG_PAD].reshape(DEG_PAD // 128, 128)
    d1r = degp[DEG_PAD:].reshape(DEG_PAD // 128, 128)

    dinvb = pl.pallas_call(
        _dexp_body,
        grid=(DEG_PAD // 1024,),
        in_specs=[
            pl.BlockSpec((8, 128), lambda b: (b, 0)),
            pl.BlockSpec((8, 128), lambda b: (b, 0)),
        ],
        out_specs=pl.BlockSpec((1024, IN_CH), lambda b: (b, 0)),
        out_shape=jax.ShapeDtypeStruct((DEG_PAD, IN_CH), jnp.float32),
    )(d0r, d1r)

    yw = pl.pallas_call(
        _yw_body,
        grid=(nb,),
        in_specs=[
            pl.BlockSpec((_RB, IN_CH), lambda b: (b, 0)),
            pl.BlockSpec((_RB, IN_CH), lambda b: (b, 0)),
        ],
        out_specs=pl.BlockSpec((_RB, IN_CH), lambda b: (b, 0)),
        out_shape=jax.ShapeDtypeStruct((N_NODES, IN_CH), jnp.float32),
    )(xw, dinvb)

    srct = edge_index[0].reshape(_NS, _NCH2, _CH2)
    dstt = edge_index[1].reshape(_NS, _NCH2, _CH2)
    sp = _msg_kernel(yw.reshape(2 * N_NODES, _HCH), srct, dstt)
    noff = _NPAD // _RB

    w1a = W1[:, :IN_CH].T
    w1b = W1[:, IN_CH:].T
    xs, xd = pl.pallas_call(
        _xsxd_body,
        grid=(nb,),
        in_specs=[
            pl.BlockSpec((_RB, _HCH), lambda b: (b, 0)),
            pl.BlockSpec((_RB, _HCH), lambda b: (b + noff, 0)),
            pl.BlockSpec((_RB, IN_CH), lambda b: (b, 0)),
            pl.BlockSpec((_RB, IN_CH), lambda b: (b, 0)),
            pl.BlockSpec((_RB, IN_CH), lambda b: (b, 0)),
            pl.BlockSpec((1, IN_CH), lambda b: (0, 0)),
            pl.BlockSpec((IN_CH, IN_CH), lambda b: (0, 0)),
            pl.BlockSpec((IN_CH, IN_CH), lambda b: (0, 0)),
            pl.BlockSpec((1, IN_CH), lambda b: (0, 0)),
        ],
        out_specs=[
            pl.BlockSpec((_RB, IN_CH), lambda b: (b, 0)),
            pl.BlockSpec((_RB, IN_CH), lambda b: (b, 0)),
        ],
        out_shape=[
            jax.ShapeDtypeStruct((N_NODES, IN_CH), jnp.float32),
            jax.ShapeDtypeStruct((N_NODES, IN_CH), jnp.float32),
        ],
    )(sp, sp, yw, dinvb, state, bc.reshape(1, IN_CH), w1a, w1b,
      b1.reshape(1, IN_CH))

    ngraph = N_NODES // NREGION
    npair = NREGION * NREGION
    xs3 = xs.reshape(ngraph, NREGION, IN_CH)
    xd3 = xd.reshape(ngraph, NREGION, IN_CH)
    c0, c1 = pl.pallas_call(
        _act_body,
        grid=(ngraph // _GB,),
        in_specs=[
            pl.BlockSpec((_GB, NREGION, IN_CH), lambda b: (b, 0, 0)),
            pl.BlockSpec((_GB, NREGION, IN_CH), lambda b: (b, 0, 0)),
            pl.BlockSpec((1, IN_CH), lambda b: (0, 0)),
            pl.BlockSpec((1, IN_CH), lambda b: (0, 0)),
            pl.BlockSpec((1, 2), lambda b: (0, 0)),
        ],
        out_specs=[
            pl.BlockSpec((_GB, npair), lambda b: (b, 0)),
            pl.BlockSpec((_GB, npair), lambda b: (b, 0)),
        ],
        out_shape=[
            jax.ShapeDtypeStruct((ngraph, npair), jnp.float32),
            jax.ShapeDtypeStruct((ngraph, npair), jnp.float32),
        ],
    )(xs3, xd3, W2[0:1, :], W2[1:2, :], b2.reshape(1, 2))

    action = pl.pallas_call(
        _ratio_body,
        grid=(5,),
        in_specs=[
            pl.BlockSpec((ngraph // 5, npair), lambda b: (b, 0)),
            pl.BlockSpec((ngraph // 5, npair), lambda b: (b, 0)),
        ],
        out_specs=pl.BlockSpec((ngraph // 5, npair), lambda b: (b, 0)),
        out_shape=jax.ShapeDtypeStruct((ngraph, npair), jnp.float32),
    )(c0, c1)
    return action.reshape(ngraph, NREGION, NREGION)


# speedup vs baseline: 1.4636x; 1.0390x over previous
import functools

import jax
import jax.numpy as jnp
from jax import lax
from jax.experimental import pallas as pl
from jax.experimental.pallas import tpu as pltpu
from jax.experimental.pallas import tpu_sc as plsc

N_NODES = 10000
N_EDGES = 320000
IN_CH = 128
NREGION = 10

_NC, _NS = 2, 16
_NW = _NC * _NS
CH = 80
NCH = N_EDGES // (_NW * CH)
DEG_PAD = 10240


def _sc_mesh():
    return plsc.VectorSubcoreMesh(
        core_axis_name="c", subcore_axis_name="s",
        num_cores=_NC, num_subcores=_NS)


@functools.partial(
    pl.kernel,
    out_type=jax.ShapeDtypeStruct((_NC * DEG_PAD,), jnp.float32),
    mesh=_sc_mesh(),
    scratch_types=[
        pltpu.VMEM((NCH, CH), jnp.int32),
        pltpu.VMEM((CH,), jnp.float32),
        pltpu.VMEM((640,), jnp.float32),
        pltpu.VMEM_SHARED((DEG_PAD,), jnp.float32),
    ],
)
def _deg_kernel(dst_hbm, degp_hbm, idx_v, ones_v, zeros_v, acc_sh):
    c = lax.axis_index("c")
    s = lax.axis_index("s")
    w = c * _NS + s
    for i in range(CH // 16):
        ones_v[pl.ds(i * 16, 16)] = jnp.ones((16,), jnp.float32)
    for i in range(640 // 16):
        zeros_v[pl.ds(i * 16, 16)] = jnp.zeros((16,), jnp.float32)
    pltpu.sync_copy(zeros_v, acc_sh.at[pl.ds(s * 640, 640)])
    pltpu.sync_copy(dst_hbm.at[w], idx_v)
    plsc.subcore_barrier()

    def body(j, carry):
        pltpu.sync_copy(ones_v, acc_sh.at[idx_v.at[j]], add=True)
        return carry

    lax.fori_loop(0, NCH, body, 0)
    plsc.subcore_barrier()
    pltpu.sync_copy(acc_sh.at[pl.ds(s * 640, 640)],
                    degp_hbm.at[pl.ds(c * DEG_PAD + s * 640, 640)])


_HCH = IN_CH // 2
_NPAD = 12800
_RPT = _NPAD // _NS
_CH2 = 80
_NCH2 = 250
_EPTP = _NCH2 * _CH2


@functools.partial(
    pl.kernel,
    out_type=jax.ShapeDtypeStruct((_NPAD, IN_CH), jnp.float32),
    mesh=_sc_mesh(),
    scratch_types=[
        pltpu.VMEM((_NCH2, _CH2), jnp.int32),
        pltpu.VMEM((_NCH2, _CH2), jnp.int32),
        pltpu.VMEM((2, _CH2, _HCH), jnp.float32),
        pltpu.VMEM((160, _HCH), jnp.float32),
        pltpu.VMEM_SHARED((_NPAD, _HCH), jnp.float32),
        pltpu.SemaphoreType.DMA,
        pltpu.SemaphoreType.DMA,
        pltpu.SemaphoreType.DMA,
        pltpu.SemaphoreType.DMA,
    ],
    compiler_params=pltpu.CompilerParams(use_tc_tiling_on_sc=False),
)
def _msg_kernel(yw_hbm, src_hbm, dst_hbm, sp_hbm,
                sidx_v, didx_v, rows_v, zrow_v, acc_sh,
                sg0, sg1, ss0, ss1):
    c = lax.axis_index("c")
    s = lax.axis_index("s")

    def zbody(j, carry):
        for i in range(_HCH // 16):
            zrow_v[j, pl.ds(i * 16, 16)] = jnp.zeros((16,), jnp.float32)
        return carry

    lax.fori_loop(0, 160, zbody, 0)
    for k in range(_RPT // 160):
        pltpu.sync_copy(zrow_v, acc_sh.at[pl.ds(s * _RPT + k * 160, 160)])
    pltpu.sync_copy(src_hbm.at[s], sidx_v)
    pltpu.sync_copy(dst_hbm.at[s], didx_v)

    coff = jnp.full((16,), 0, jnp.int32) + c

    def ibody(j, carry):
        for i in range(_CH2 // 16):
            v = sidx_v[j, pl.ds(i * 16, 16)]
            sidx_v[j, pl.ds(i * 16, 16)] = v * 2 + coff
        return carry

    lax.fori_loop(0, _NCH2, ibody, 0)
    plsc.subcore_barrier()

    nit = _NCH2 // 2
    pltpu.async_copy(yw_hbm.at[sidx_v.at[0]], rows_v.at[0], sg0)
    pltpu.async_copy(yw_hbm.at[sidx_v.at[1]], rows_v.at[1], sg1)

    def body(t, carry):
        j0 = 2 * t
        pltpu.make_async_copy(
            yw_hbm.at[sidx_v.at[j0]], rows_v.at[0], sg0).wait()
        pltpu.async_copy(rows_v.at[0], acc_sh.at[didx_v.at[j0]], ss0,
                         add=True)

        @pl.when(t >= 1)
        def _():
            pltpu.make_async_copy(
                rows_v.at[0], acc_sh.at[didx_v.at[j0]], ss0).wait()

        @pl.when(t < nit - 1)
        def _():
            pltpu.async_copy(yw_hbm.at[sidx_v.at[j0 + 2]], rows_v.at[0], sg0)

        pltpu.make_async_copy(
            yw_hbm.at[sidx_v.at[j0 + 1]], rows_v.at[1], sg1).wait()
        pltpu.async_copy(rows_v.at[1], acc_sh.at[didx_v.at[j0 + 1]], ss1,
                         add=True)

        @pl.when(t >= 1)
        def _():
            pltpu.make_async_copy(
                rows_v.at[1], acc_sh.at[didx_v.at[j0 + 1]], ss1).wait()

        @pl.when(t < nit - 1)
        def _():
            pltpu.async_copy(yw_hbm.at[sidx_v.at[j0 + 3]], rows_v.at[1], sg1)

        return carry

    lax.fori_loop(0, nit, body, 0)
    pltpu.make_async_copy(
        rows_v.at[0], acc_sh.at[didx_v.at[_NCH2 - 2]], ss0).wait()
    pltpu.make_async_copy(
        rows_v.at[1], acc_sh.at[didx_v.at[_NCH2 - 1]], ss1).wait()
    plsc.subcore_barrier()
    for k in range(_RPT // 160):
        pltpu.sync_copy(
            acc_sh.at[pl.ds(s * _RPT + k * 160, 160)],
            sp_hbm.at[pl.ds(s * _RPT + k * 160, 160), pl.ds(c * _HCH, _HCH)])


_RB = 400


def _xw_body(state_ref, wc_ref, xw_ref):
    xw_ref[...] = jnp.dot(state_ref[...], wc_ref[...],
                          preferred_element_type=jnp.float32)


def _dexp_body(d0_ref, d1_ref, dinvb_ref):
    dinv = lax.rsqrt(d0_ref[...] + d1_ref[...] + 1.0)
    ones = jnp.ones((1, IN_CH), jnp.float32)
    for r in range(8):
        dinvb_ref[r * 128:(r + 1) * 128, :] = lax.dot_general(
            dinv[r:r + 1, :], ones, (((0,), (0,)), ((), ())),
            preferred_element_type=jnp.float32)


def _yw_body(xw_ref, dv_ref, yw_ref):
    yw_ref[...] = xw_ref[...] * dv_ref[...]


def _xsxd_body(s_ref, yw_ref, dv_ref, state_ref, bc_ref,
               w1a_ref, w1b_ref, b1_ref, xs_ref, xd_ref):
    gcn = dv_ref[...] * (s_ref[...] + yw_ref[...]) + bc_ref[...]
    x = jnp.maximum(gcn, 0.0) + state_ref[...]
    xs_ref[...] = jnp.dot(x, w1a_ref[...],
                          preferred_element_type=jnp.float32) + b1_ref[...]
    xd_ref[...] = jnp.dot(x, w1b_ref[...],
                          preferred_element_type=jnp.float32)


_GB = 40


def _act_body(xs_ref, xd_ref, w20_ref, w21_ref, b2_ref, c0_ref, c1_ref):
    xd = xd_ref[...]
    w20 = w20_ref[...][None]
    w21 = w21_ref[...][None]
    b20 = b2_ref[0:1, 0:1]
    b21 = b2_ref[0:1, 1:2]
    for i in range(NREGION):
        z = xs_ref[:, i:i + 1, :] + xd
        h = jnp.where(z >= 0, z, 0.01 * z)
        c0_ref[:, i * 10:(i + 1) * 10] = jnp.sum(h * w20, axis=-1) + b20
        c1_ref[:, i * 10:(i + 1) * 10] = jnp.sum(h * w21, axis=-1) + b21


def _ratio_body(c0_ref, c1_ref, out_ref):
    def softplus(v):
        return jnp.maximum(v, 0.0) + jnp.log1p(jnp.exp(-jnp.abs(v)))

    p0 = softplus(c0_ref[...])
    p1 = softplus(c1_ref[...])
    a = p0 / (p0 + p1 + 1e-10)
    out_ref[...] = jnp.where(a < 0, 0.0, a)


def kernel(state, edge_index, edges, Wc, bc, W1, b1, W2, b2):
    del edges
    src2 = edge_index[0].reshape(_NW, NCH, CH)
    dst2 = edge_index[1].reshape(_NW, NCH, CH)

    nb = N_NODES // _RB
    xw = pl.pallas_call(
        _xw_body,
        grid=(nb,),
        in_specs=[
            pl.BlockSpec((_RB, IN_CH), lambda b: (b, 0)),
            pl.BlockSpec((IN_CH, IN_CH), lambda b: (0, 0)),
        ],
        out_specs=pl.BlockSpec((_RB, IN_CH), lambda b: (b, 0)),
        out_shape=jax.ShapeDtypeStruct((N_NODES, IN_CH), jnp.float32),
    )(state, Wc)

    degp = _deg_kernel(dst2)
    d0r = degp[:DEG_PAD].reshape(DEG_PAD // 128, 128)
    d1r = degp[DEG_PAD:].reshape(DEG_PAD // 128, 128)

    dinvb = pl.pallas_call(
        _dexp_body,
        grid=(DEG_PAD // 1024,),
        in_specs=[
            pl.BlockSpec((8, 128), lambda b: (b, 0)),
            pl.BlockSpec((8, 128), lambda b: (b, 0)),
        ],
        out_specs=pl.BlockSpec((1024, IN_CH), lambda b: (b, 0)),
        out_shape=jax.ShapeDtypeStruct((DEG_PAD, IN_CH), jnp.float32),
    )(d0r, d1r)

    yw = pl.pallas_call(
        _yw_body,
        grid=(nb,),
        in_specs=[
            pl.BlockSpec((_RB, IN_CH), lambda b: (b, 0)),
            pl.BlockSpec((_RB, IN_CH), lambda b: (b, 0)),
        ],
        out_specs=pl.BlockSpec((_RB, IN_CH), lambda b: (b, 0)),
        out_shape=jax.ShapeDtypeStruct((N_NODES, IN_CH), jnp.float32),
    )(xw, dinvb)

    srct = edge_index[0].reshape(_NS, _NCH2, _CH2)
    dstt = edge_index[1].reshape(_NS, _NCH2, _CH2)
    sp = _msg_kernel(yw.reshape(2 * N_NODES, _HCH), srct, dstt)

    w1a = W1[:, :IN_CH].T
    w1b = W1[:, IN_CH:].T
    xs, xd = pl.pallas_call(
        _xsxd_body,
        grid=(nb,),
        in_specs=[
            pl.BlockSpec((_RB, IN_CH), lambda b: (b, 0)),
            pl.BlockSpec((_RB, IN_CH), lambda b: (b, 0)),
            pl.BlockSpec((_RB, IN_CH), lambda b: (b, 0)),
            pl.BlockSpec((_RB, IN_CH), lambda b: (b, 0)),
            pl.BlockSpec((1, IN_CH), lambda b: (0, 0)),
            pl.BlockSpec((IN_CH, IN_CH), lambda b: (0, 0)),
            pl.BlockSpec((IN_CH, IN_CH), lambda b: (0, 0)),
            pl.BlockSpec((1, IN_CH), lambda b: (0, 0)),
        ],
        out_specs=[
            pl.BlockSpec((_RB, IN_CH), lambda b: (b, 0)),
            pl.BlockSpec((_RB, IN_CH), lambda b: (b, 0)),
        ],
        out_shape=[
            jax.ShapeDtypeStruct((N_NODES, IN_CH), jnp.float32),
            jax.ShapeDtypeStruct((N_NODES, IN_CH), jnp.float32),
        ],
    )(sp, yw, dinvb, state, bc.reshape(1, IN_CH), w1a, w1b,
      b1.reshape(1, IN_CH))

    ngraph = N_NODES // NREGION
    npair = NREGION * NREGION
    xs3 = xs.reshape(ngraph, NREGION, IN_CH)
    xd3 = xd.reshape(ngraph, NREGION, IN_CH)
    c0, c1 = pl.pallas_call(
        _act_body,
        grid=(ngraph // _GB,),
        in_specs=[
            pl.BlockSpec((_GB, NREGION, IN_CH), lambda b: (b, 0, 0)),
            pl.BlockSpec((_GB, NREGION, IN_CH), lambda b: (b, 0, 0)),
            pl.BlockSpec((1, IN_CH), lambda b: (0, 0)),
            pl.BlockSpec((1, IN_CH), lambda b: (0, 0)),
            pl.BlockSpec((1, 2), lambda b: (0, 0)),
        ],
        out_specs=[
            pl.BlockSpec((_GB, npair), lambda b: (b, 0)),
            pl.BlockSpec((_GB, npair), lambda b: (b, 0)),
        ],
        out_shape=[
            jax.ShapeDtypeStruct((ngraph, npair), jnp.float32),
            jax.ShapeDtypeStruct((ngraph, npair), jnp.float32),
        ],
    )(xs3, xd3, W2[0:1, :], W2[1:2, :], b2.reshape(1, 2))

    action = pl.pallas_call(
        _ratio_body,
        grid=(5,),
        in_specs=[
            pl.BlockSpec((ngraph // 5, npair), lambda b: (b, 0)),
            pl.BlockSpec((ngraph // 5, npair), lambda b: (b, 0)),
        ],
        out_specs=pl.BlockSpec((ngraph // 5, npair), lambda b: (b, 0)),
        out_shape=jax.ShapeDtypeStruct((ngraph, npair), jnp.float32),
    )(c0, c1)
    return action.reshape(ngraph, NREGION, NREGION)


# speedup vs baseline: 1.5119x; 1.0330x over previous
import functools

import jax
import jax.numpy as jnp
from jax import lax
from jax.experimental import pallas as pl
from jax.experimental.pallas import tpu as pltpu
from jax.experimental.pallas import tpu_sc as plsc

N_NODES = 10000
N_EDGES = 320000
IN_CH = 128
NREGION = 10

_NC, _NS = 2, 16
_NW = _NC * _NS
CH = 80
NCH = N_EDGES // (_NW * CH)
DEG_PAD = 10240


def _sc_mesh():
    return plsc.VectorSubcoreMesh(
        core_axis_name="c", subcore_axis_name="s",
        num_cores=_NC, num_subcores=_NS)


@functools.partial(
    pl.kernel,
    out_type=jax.ShapeDtypeStruct((_NC * DEG_PAD,), jnp.float32),
    mesh=_sc_mesh(),
    scratch_types=[
        pltpu.VMEM((NCH, CH), jnp.int32),
        pltpu.VMEM((CH,), jnp.float32),
        pltpu.VMEM((640,), jnp.float32),
        pltpu.VMEM_SHARED((DEG_PAD,), jnp.float32),
    ],
)
def _deg_kernel(dst_hbm, degp_hbm, idx_v, ones_v, zeros_v, acc_sh):
    c = lax.axis_index("c")
    s = lax.axis_index("s")
    w = c * _NS + s
    for i in range(CH // 16):
        ones_v[pl.ds(i * 16, 16)] = jnp.ones((16,), jnp.float32)
    for i in range(640 // 16):
        zeros_v[pl.ds(i * 16, 16)] = jnp.zeros((16,), jnp.float32)
    pltpu.sync_copy(zeros_v, acc_sh.at[pl.ds(s * 640, 640)])
    pltpu.sync_copy(dst_hbm.at[w], idx_v)
    plsc.subcore_barrier()

    def body(j, carry):
        pltpu.sync_copy(ones_v, acc_sh.at[idx_v.at[j]], add=True)
        return carry

    lax.fori_loop(0, NCH, body, 0)
    plsc.subcore_barrier()
    pltpu.sync_copy(acc_sh.at[pl.ds(s * 640, 640)],
                    degp_hbm.at[pl.ds(c * DEG_PAD + s * 640, 640)])


_HCH = IN_CH // 2
_NPAD = 12800
_RPT = _NPAD // _NS
_CH2 = 80
_NCH2 = 250
_EPTP = _NCH2 * _CH2


@functools.partial(
    pl.kernel,
    out_type=jax.ShapeDtypeStruct((_NPAD, IN_CH), jnp.float32),
    mesh=_sc_mesh(),
    scratch_types=[
        pltpu.VMEM((_NCH2, _CH2), jnp.int32),
        pltpu.VMEM((_NCH2, _CH2), jnp.int32),
        pltpu.VMEM((2, _CH2, _HCH), jnp.float32),
        pltpu.VMEM((160, _HCH), jnp.float32),
        pltpu.VMEM_SHARED((_NPAD, _HCH), jnp.float32),
        pltpu.SemaphoreType.DMA,
        pltpu.SemaphoreType.DMA,
        pltpu.SemaphoreType.DMA,
        pltpu.SemaphoreType.DMA,
    ],
    compiler_params=pltpu.CompilerParams(use_tc_tiling_on_sc=False),
)
def _msg_kernel(yw_hbm, src_hbm, dst_hbm, sp_hbm,
                sidx_v, didx_v, rows_v, zrow_v, acc_sh,
                sg0, sg1, ss0, ss1):
    c = lax.axis_index("c")
    s = lax.axis_index("s")

    def zbody(j, carry):
        for i in range(_HCH // 16):
            zrow_v[j, pl.ds(i * 16, 16)] = jnp.zeros((16,), jnp.float32)
        return carry

    lax.fori_loop(0, 160, zbody, 0)
    for k in range(_RPT // 160):
        pltpu.sync_copy(zrow_v, acc_sh.at[pl.ds(s * _RPT + k * 160, 160)])
    pltpu.sync_copy(src_hbm.at[s], sidx_v)
    pltpu.sync_copy(dst_hbm.at[s], didx_v)

    coff = jnp.full((16,), 0, jnp.int32) + c

    def ibody(j, carry):
        for i in range(_CH2 // 16):
            v = sidx_v[j, pl.ds(i * 16, 16)]
            sidx_v[j, pl.ds(i * 16, 16)] = v * 2 + coff
        return carry

    lax.fori_loop(0, _NCH2, ibody, 0)
    plsc.subcore_barrier()

    nit = _NCH2 // 2
    pltpu.async_copy(yw_hbm.at[sidx_v.at[0]], rows_v.at[0], sg0)
    pltpu.async_copy(yw_hbm.at[sidx_v.at[1]], rows_v.at[1], sg1)

    def body(t, carry):
        j0 = 2 * t
        pltpu.make_async_copy(
            yw_hbm.at[sidx_v.at[j0]], rows_v.at[0], sg0).wait()
        pltpu.async_copy(rows_v.at[0], acc_sh.at[didx_v.at[j0]], ss0,
                         add=True)

        @pl.when(t >= 1)
        def _():
            pltpu.make_async_copy(
                rows_v.at[0], acc_sh.at[didx_v.at[j0]], ss0).wait()

        @pl.when(t < nit - 1)
        def _():
            pltpu.async_copy(yw_hbm.at[sidx_v.at[j0 + 2]], rows_v.at[0], sg0)

        pltpu.make_async_copy(
            yw_hbm.at[sidx_v.at[j0 + 1]], rows_v.at[1], sg1).wait()
        pltpu.async_copy(rows_v.at[1], acc_sh.at[didx_v.at[j0 + 1]], ss1,
                         add=True)

        @pl.when(t >= 1)
        def _():
            pltpu.make_async_copy(
                rows_v.at[1], acc_sh.at[didx_v.at[j0 + 1]], ss1).wait()

        @pl.when(t < nit - 1)
        def _():
            pltpu.async_copy(yw_hbm.at[sidx_v.at[j0 + 3]], rows_v.at[1], sg1)

        return carry

    lax.fori_loop(0, nit, body, 0)
    pltpu.make_async_copy(
        rows_v.at[0], acc_sh.at[didx_v.at[_NCH2 - 2]], ss0).wait()
    pltpu.make_async_copy(
        rows_v.at[1], acc_sh.at[didx_v.at[_NCH2 - 1]], ss1).wait()
    plsc.subcore_barrier()
    for k in range(_RPT // 160):
        pltpu.sync_copy(
            acc_sh.at[pl.ds(s * _RPT + k * 160, 160)],
            sp_hbm.at[pl.ds(s * _RPT + k * 160, 160), pl.ds(c * _HCH, _HCH)])


_RB = 400


def _xw_body(state_ref, wc_ref, xw_ref):
    xw_ref[...] = jnp.dot(state_ref[...], wc_ref[...],
                          preferred_element_type=jnp.float32)


def _dexp_body(d0_ref, d1_ref, dinvb_ref):
    dinv = lax.rsqrt(d0_ref[...] + d1_ref[...] + 1.0)
    ones = jnp.ones((1, IN_CH), jnp.float32)
    for r in range(8):
        dinvb_ref[r * 128:(r + 1) * 128, :] = lax.dot_general(
            dinv[r:r + 1, :], ones, (((0,), (0,)), ((), ())),
            preferred_element_type=jnp.float32)


def _yw_body(xw_ref, dv_ref, yw_ref):
    yw_ref[...] = xw_ref[...] * dv_ref[...]


def _xsxd_body(s_ref, yw_ref, dv_ref, state_ref, bc_ref,
               w1a_ref, w1b_ref, b1_ref, xs_ref, xd_ref):
    gcn = dv_ref[...] * (s_ref[...] + yw_ref[...]) + bc_ref[...]
    x = jnp.maximum(gcn, 0.0) + state_ref[...]
    xs_ref[...] = jnp.dot(x, w1a_ref[...],
                          preferred_element_type=jnp.float32) + b1_ref[...]
    xd_ref[...] = jnp.dot(x, w1b_ref[...],
                          preferred_element_type=jnp.float32)


_GB = 40


def _act_body(xs_ref, xd_ref, w20_ref, w21_ref, b2_ref, c0_ref, c1_ref):
    xd = xd_ref[...]
    w20 = w20_ref[...][None]
    w21 = w21_ref[...][None]
    b20 = b2_ref[0:1, 0:1]
    b21 = b2_ref[0:1, 1:2]
    for i0, w in ((0, 4), (4, 4), (8, 2)):
        z = jnp.concatenate(
            [xs_ref[:, i:i + 1, :] + xd for i in range(i0, i0 + w)], axis=1)
        h = jnp.where(z >= 0, z, 0.01 * z)
        c0_ref[:, i0 * 10:(i0 + w) * 10] = jnp.sum(h * w20, axis=-1) + b20
        c1_ref[:, i0 * 10:(i0 + w) * 10] = jnp.sum(h * w21, axis=-1) + b21


def _ratio_body(c0_ref, c1_ref, out_ref):
    def softplus(v):
        return jnp.maximum(v, 0.0) + jnp.log1p(jnp.exp(-jnp.abs(v)))

    p0 = softplus(c0_ref[...])
    p1 = softplus(c1_ref[...])
    a = p0 / (p0 + p1 + 1e-10)
    out_ref[...] = jnp.where(a < 0, 0.0, a)


def kernel(state, edge_index, edges, Wc, bc, W1, b1, W2, b2):
    del edges
    src2 = edge_index[0].reshape(_NW, NCH, CH)
    dst2 = edge_index[1].reshape(_NW, NCH, CH)

    nb = N_NODES // _RB
    xw = pl.pallas_call(
        _xw_body,
        grid=(nb,),
        in_specs=[
            pl.BlockSpec((_RB, IN_CH), lambda b: (b, 0)),
            pl.BlockSpec((IN_CH, IN_CH), lambda b: (0, 0)),
        ],
        out_specs=pl.BlockSpec((_RB, IN_CH), lambda b: (b, 0)),
        out_shape=jax.ShapeDtypeStruct((N_NODES, IN_CH), jnp.float32),
    )(state, Wc)

    degp = _deg_kernel(dst2)
    d0r = degp[:DEG_PAD].reshape(DEG_PAD // 128, 128)
    d1r = degp[DEG_PAD:].reshape(DEG_PAD // 128, 128)

    dinvb = pl.pallas_call(
        _dexp_body,
        grid=(DEG_PAD // 1024,),
        in_specs=[
            pl.BlockSpec((8, 128), lambda b: (b, 0)),
            pl.BlockSpec((8, 128), lambda b: (b, 0)),
        ],
        out_specs=pl.BlockSpec((1024, IN_CH), lambda b: (b, 0)),
        out_shape=jax.ShapeDtypeStruct((DEG_PAD, IN_CH), jnp.float32),
    )(d0r, d1r)

    yw = pl.pallas_call(
        _yw_body,
        grid=(nb,),
        in_specs=[
            pl.BlockSpec((_RB, IN_CH), lambda b: (b, 0)),
            pl.BlockSpec((_RB, IN_CH), lambda b: (b, 0)),
        ],
        out_specs=pl.BlockSpec((_RB, IN_CH), lambda b: (b, 0)),
        out_shape=jax.ShapeDtypeStruct((N_NODES, IN_CH), jnp.float32),
    )(xw, dinvb)

    srct = edge_index[0].reshape(_NS, _NCH2, _CH2)
    dstt = edge_index[1].reshape(_NS, _NCH2, _CH2)
    sp = _msg_kernel(yw.reshape(2 * N_NODES, _HCH), srct, dstt)

    w1a = W1[:, :IN_CH].T
    w1b = W1[:, IN_CH:].T
    xs, xd = pl.pallas_call(
        _xsxd_body,
        grid=(nb,),
        in_specs=[
            pl.BlockSpec((_RB, IN_CH), lambda b: (b, 0)),
            pl.BlockSpec((_RB, IN_CH), lambda b: (b, 0)),
            pl.BlockSpec((_RB, IN_CH), lambda b: (b, 0)),
            pl.BlockSpec((_RB, IN_CH), lambda b: (b, 0)),
            pl.BlockSpec((1, IN_CH), lambda b: (0, 0)),
            pl.BlockSpec((IN_CH, IN_CH), lambda b: (0, 0)),
            pl.BlockSpec((IN_CH, IN_CH), lambda b: (0, 0)),
            pl.BlockSpec((1, IN_CH), lambda b: (0, 0)),
        ],
        out_specs=[
            pl.BlockSpec((_RB, IN_CH), lambda b: (b, 0)),
            pl.BlockSpec((_RB, IN_CH), lambda b: (b, 0)),
        ],
        out_shape=[
            jax.ShapeDtypeStruct((N_NODES, IN_CH), jnp.float32),
            jax.ShapeDtypeStruct((N_NODES, IN_CH), jnp.float32),
        ],
    )(sp, yw, dinvb, state, bc.reshape(1, IN_CH), w1a, w1b,
      b1.reshape(1, IN_CH))

    ngraph = N_NODES // NREGION
    npair = NREGION * NREGION
    xs3 = xs.reshape(ngraph, NREGION, IN_CH)
    xd3 = xd.reshape(ngraph, NREGION, IN_CH)
    c0, c1 = pl.pallas_call(
        _act_body,
        grid=(ngraph // _GB,),
        in_specs=[
            pl.BlockSpec((_GB, NREGION, IN_CH), lambda b: (b, 0, 0)),
            pl.BlockSpec((_GB, NREGION, IN_CH), lambda b: (b, 0, 0)),
            pl.BlockSpec((1, IN_CH), lambda b: (0, 0)),
            pl.BlockSpec((1, IN_CH), lambda b: (0, 0)),
            pl.BlockSpec((1, 2), lambda b: (0, 0)),
        ],
        out_specs=[
            pl.BlockSpec((_GB, npair), lambda b: (b, 0)),
            pl.BlockSpec((_GB, npair), lambda b: (b, 0)),
        ],
        out_shape=[
            jax.ShapeDtypeStruct((ngraph, npair), jnp.float32),
            jax.ShapeDtypeStruct((ngraph, npair), jnp.float32),
        ],
    )(xs3, xd3, W2[0:1, :], W2[1:2, :], b2.reshape(1, 2))

    action = pl.pallas_call(
        _ratio_body,
        grid=(5,),
        in_specs=[
            pl.BlockSpec((ngraph // 5, npair), lambda b: (b, 0)),
            pl.BlockSpec((ngraph // 5, npair), lambda b: (b, 0)),
        ],
        out_specs=pl.BlockSpec((ngraph // 5, npair), lambda b: (b, 0)),
        out_shape=jax.ShapeDtypeStruct((ngraph, npair), jnp.float32),
    )(c0, c1)
    return action.reshape(ngraph, NREGION, NREGION)


# speedup vs baseline: 1.5575x; 1.0301x over previous
import functools

import jax
import jax.numpy as jnp
from jax import lax
from jax.experimental import pallas as pl
from jax.experimental.pallas import tpu as pltpu
from jax.experimental.pallas import tpu_sc as plsc

N_NODES = 10000
N_EDGES = 320000
IN_CH = 128
NREGION = 10

_NC, _NS = 2, 16
_NW = _NC * _NS
CH = 80
NCH = N_EDGES // (_NW * CH)
DEG_PAD = 10240


def _sc_mesh():
    return plsc.VectorSubcoreMesh(
        core_axis_name="c", subcore_axis_name="s",
        num_cores=_NC, num_subcores=_NS)


@functools.partial(
    pl.kernel,
    out_type=jax.ShapeDtypeStruct((_NC * DEG_PAD,), jnp.float32),
    mesh=_sc_mesh(),
    scratch_types=[
        pltpu.VMEM((NCH, CH), jnp.int32),
        pltpu.VMEM((CH,), jnp.float32),
        pltpu.VMEM((640,), jnp.float32),
        pltpu.VMEM_SHARED((DEG_PAD,), jnp.float32),
    ],
)
def _deg_kernel(dst_hbm, degp_hbm, idx_v, ones_v, zeros_v, acc_sh):
    c = lax.axis_index("c")
    s = lax.axis_index("s")
    w = c * _NS + s
    for i in range(CH // 16):
        ones_v[pl.ds(i * 16, 16)] = jnp.ones((16,), jnp.float32)
    for i in range(640 // 16):
        zeros_v[pl.ds(i * 16, 16)] = jnp.zeros((16,), jnp.float32)
    pltpu.sync_copy(zeros_v, acc_sh.at[pl.ds(s * 640, 640)])
    pltpu.sync_copy(dst_hbm.at[w], idx_v)
    plsc.subcore_barrier()

    def body(j, carry):
        pltpu.sync_copy(ones_v, acc_sh.at[idx_v.at[j]], add=True)
        return carry

    lax.fori_loop(0, NCH, body, 0)
    plsc.subcore_barrier()
    pltpu.sync_copy(acc_sh.at[pl.ds(s * 640, 640)],
                    degp_hbm.at[pl.ds(c * DEG_PAD + s * 640, 640)])


_HCH = IN_CH // 2
_NPAD = 12800
_RPT = _NPAD // _NS
_CH2 = 80
_NCH2 = 250
_EPTP = _NCH2 * _CH2


@functools.partial(
    pl.kernel,
    out_type=jax.ShapeDtypeStruct((_NPAD, IN_CH), jnp.float32),
    mesh=_sc_mesh(),
    scratch_types=[
        pltpu.VMEM((_NCH2, _CH2), jnp.int32),
        pltpu.VMEM((_NCH2, _CH2), jnp.int32),
        pltpu.VMEM((2, _CH2, _HCH), jnp.float32),
        pltpu.VMEM((160, _HCH), jnp.float32),
        pltpu.VMEM_SHARED((_NPAD, _HCH), jnp.float32),
        pltpu.SemaphoreType.DMA,
        pltpu.SemaphoreType.DMA,
        pltpu.SemaphoreType.DMA,
        pltpu.SemaphoreType.DMA,
    ],
    compiler_params=pltpu.CompilerParams(use_tc_tiling_on_sc=False),
)
def _msg_kernel(yw_hbm, src_hbm, dst_hbm, sp_hbm,
                sidx_v, didx_v, rows_v, zrow_v, acc_sh,
                sg0, sg1, ss0, ss1):
    c = lax.axis_index("c")
    s = lax.axis_index("s")

    def zbody(j, carry):
        for i in range(_HCH // 16):
            zrow_v[j, pl.ds(i * 16, 16)] = jnp.zeros((16,), jnp.float32)
        return carry

    lax.fori_loop(0, 160, zbody, 0)
    for k in range(_RPT // 160):
        pltpu.sync_copy(zrow_v, acc_sh.at[pl.ds(s * _RPT + k * 160, 160)])
    pltpu.sync_copy(src_hbm.at[s], sidx_v)
    pltpu.sync_copy(dst_hbm.at[s], didx_v)

    coff = jnp.full((16,), 0, jnp.int32) + c

    def ibody(j, carry):
        for i in range(_CH2 // 16):
            v = sidx_v[j, pl.ds(i * 16, 16)]
            sidx_v[j, pl.ds(i * 16, 16)] = v * 2 + coff
        return carry

    lax.fori_loop(0, _NCH2, ibody, 0)
    plsc.subcore_barrier()

    nit = _NCH2 // 2
    pltpu.async_copy(yw_hbm.at[sidx_v.at[0]], rows_v.at[0], sg0)
    pltpu.async_copy(yw_hbm.at[sidx_v.at[1]], rows_v.at[1], sg1)

    def body(t, carry):
        j0 = 2 * t
        pltpu.make_async_copy(
            yw_hbm.at[sidx_v.at[j0]], rows_v.at[0], sg0).wait()
        pltpu.async_copy(rows_v.at[0], acc_sh.at[didx_v.at[j0]], ss0,
                         add=True)

        @pl.when(t >= 1)
        def _():
            pltpu.make_async_copy(
                rows_v.at[0], acc_sh.at[didx_v.at[j0]], ss0).wait()

        @pl.when(t < nit - 1)
        def _():
            pltpu.async_copy(yw_hbm.at[sidx_v.at[j0 + 2]], rows_v.at[0], sg0)

        pltpu.make_async_copy(
            yw_hbm.at[sidx_v.at[j0 + 1]], rows_v.at[1], sg1).wait()
        pltpu.async_copy(rows_v.at[1], acc_sh.at[didx_v.at[j0 + 1]], ss1,
                         add=True)

        @pl.when(t >= 1)
        def _():
            pltpu.make_async_copy(
                rows_v.at[1], acc_sh.at[didx_v.at[j0 + 1]], ss1).wait()

        @pl.when(t < nit - 1)
        def _():
            pltpu.async_copy(yw_hbm.at[sidx_v.at[j0 + 3]], rows_v.at[1], sg1)

        return carry

    lax.fori_loop(0, nit, body, 0)
    pltpu.make_async_copy(
        rows_v.at[0], acc_sh.at[didx_v.at[_NCH2 - 2]], ss0).wait()
    pltpu.make_async_copy(
        rows_v.at[1], acc_sh.at[didx_v.at[_NCH2 - 1]], ss1).wait()
    plsc.subcore_barrier()
    for k in range(_RPT // 160):
        pltpu.sync_copy(
            acc_sh.at[pl.ds(s * _RPT + k * 160, 160)],
            sp_hbm.at[pl.ds(s * _RPT + k * 160, 160), pl.ds(c * _HCH, _HCH)])


_RB = 400


def _xw_body(state_ref, wc_ref, xw_ref):
    xw_ref[...] = jnp.dot(state_ref[...], wc_ref[...],
                          preferred_element_type=jnp.float32)


def _dexp_body(d0_ref, d1_ref, dinvb_ref):
    dinv = lax.rsqrt(d0_ref[...] + d1_ref[...] + 1.0)
    ones = jnp.ones((1, IN_CH), jnp.float32)
    for r in range(8):
        dinvb_ref[r * 128:(r + 1) * 128, :] = lax.dot_general(
            dinv[r:r + 1, :], ones, (((0,), (0,)), ((), ())),
            preferred_element_type=jnp.float32)


def _yw_body(xw_ref, dv_ref, yw_ref):
    yw_ref[...] = xw_ref[...] * dv_ref[...]


def _xsxd_body(s_ref, yw_ref, dv_ref, state_ref, bc_ref,
               w1a_ref, w1b_ref, b1_ref, xs_ref, xd_ref):
    gcn = dv_ref[...] * (s_ref[...] + yw_ref[...]) + bc_ref[...]
    x = jnp.maximum(gcn, 0.0) + state_ref[...]
    xs = jnp.dot(x, w1a_ref[...],
                 preferred_element_type=jnp.float32) + b1_ref[...]
    xd = jnp.dot(x, w1b_ref[...], preferred_element_type=jnp.float32)
    xs_ref[...] = xs.reshape(_RB // NREGION, NREGION, IN_CH)
    xd_ref[...] = xd.reshape(_RB // NREGION, NREGION, IN_CH)


_GB = 40


def _act_body(xs_ref, xd_ref, w20_ref, w21_ref, b2_ref, c0_ref, c1_ref):
    xd = xd_ref[...]
    w20 = w20_ref[...][None]
    w21 = w21_ref[...][None]
    b20 = b2_ref[0:1, 0:1]
    b21 = b2_ref[0:1, 1:2]
    for i0, w in ((0, 4), (4, 4), (8, 2)):
        z = jnp.concatenate(
            [xs_ref[:, i:i + 1, :] + xd for i in range(i0, i0 + w)], axis=1)
        h = jnp.where(z >= 0, z, 0.01 * z)
        c0_ref[:, i0 * 10:(i0 + w) * 10] = jnp.sum(h * w20, axis=-1) + b20
        c1_ref[:, i0 * 10:(i0 + w) * 10] = jnp.sum(h * w21, axis=-1) + b21


def _ratio_body(c0_ref, c1_ref, out_ref):
    def softplus(v):
        return jnp.maximum(v, 0.0) + jnp.log1p(jnp.exp(-jnp.abs(v)))

    p0 = softplus(c0_ref[...])
    p1 = softplus(c1_ref[...])
    a = p0 / (p0 + p1 + 1e-10)
    out_ref[...] = jnp.where(a < 0, 0.0, a)


def kernel(state, edge_index, edges, Wc, bc, W1, b1, W2, b2):
    del edges
    src2 = edge_index[0].reshape(_NW, NCH, CH)
    dst2 = edge_index[1].reshape(_NW, NCH, CH)

    nb = N_NODES // _RB
    xw = pl.pallas_call(
        _xw_body,
        grid=(nb,),
        in_specs=[
            pl.BlockSpec((_RB, IN_CH), lambda b: (b, 0)),
            pl.BlockSpec((IN_CH, IN_CH), lambda b: (0, 0)),
        ],
        out_specs=pl.BlockSpec((_RB, IN_CH), lambda b: (b, 0)),
        out_shape=jax.ShapeDtypeStruct((N_NODES, IN_CH), jnp.float32),
    )(state, Wc)

    degp = _deg_kernel(dst2)
    d0r = degp[:DEG_PAD].reshape(DEG_PAD // 128, 128)
    d1r = degp[DEG_PAD:].reshape(DEG_PAD // 128, 128)

    dinvb = pl.pallas_call(
        _dexp_body,
        grid=(DEG_PAD // 1024,),
        in_specs=[
            pl.BlockSpec((8, 128), lambda b: (b, 0)),
            pl.BlockSpec((8, 128), lambda b: (b, 0)),
        ],
        out_specs=pl.BlockSpec((1024, IN_CH), lambda b: (b, 0)),
        out_shape=jax.ShapeDtypeStruct((DEG_PAD, IN_CH), jnp.float32),
    )(d0r, d1r)

    yw = pl.pallas_call(
        _yw_body,
        grid=(nb,),
        in_specs=[
            pl.BlockSpec((_RB, IN_CH), lambda b: (b, 0)),
            pl.BlockSpec((_RB, IN_CH), lambda b: (b, 0)),
        ],
        out_specs=pl.BlockSpec((_RB, IN_CH), lambda b: (b, 0)),
        out_shape=jax.ShapeDtypeStruct((N_NODES, IN_CH), jnp.float32),
    )(xw, dinvb)

    srct = edge_index[0].reshape(_NS, _NCH2, _CH2)
    dstt = edge_index[1].reshape(_NS, _NCH2, _CH2)
    sp = _msg_kernel(yw.reshape(2 * N_NODES, _HCH), srct, dstt)

    w1a = W1[:, :IN_CH].T
    w1b = W1[:, IN_CH:].T
    xs, xd = pl.pallas_call(
        _xsxd_body,
        grid=(nb,),
        in_specs=[
            pl.BlockSpec((_RB, IN_CH), lambda b: (b, 0)),
            pl.BlockSpec((_RB, IN_CH), lambda b: (b, 0)),
            pl.BlockSpec((_RB, IN_CH), lambda b: (b, 0)),
            pl.BlockSpec((_RB, IN_CH), lambda b: (b, 0)),
            pl.BlockSpec((1, IN_CH), lambda b: (0, 0)),
            pl.BlockSpec((IN_CH, IN_CH), lambda b: (0, 0)),
            pl.BlockSpec((IN_CH, IN_CH), lambda b: (0, 0)),
            pl.BlockSpec((1, IN_CH), lambda b: (0, 0)),
        ],
        out_specs=[
            pl.BlockSpec((_RB // NREGION, NREGION, IN_CH), lambda b: (b, 0, 0)),
            pl.BlockSpec((_RB // NREGION, NREGION, IN_CH), lambda b: (b, 0, 0)),
        ],
        out_shape=[
            jax.ShapeDtypeStruct((N_NODES // NREGION, NREGION, IN_CH),
                                 jnp.float32),
            jax.ShapeDtypeStruct((N_NODES // NREGION, NREGION, IN_CH),
                                 jnp.float32),
        ],
    )(sp, yw, dinvb, state, bc.reshape(1, IN_CH), w1a, w1b,
      b1.reshape(1, IN_CH))

    ngraph = N_NODES // NREGION
    npair = NREGION * NREGION
    xs3 = xs
    xd3 = xd
    c0, c1 = pl.pallas_call(
        _act_body,
        grid=(ngraph // _GB,),
        in_specs=[
            pl.BlockSpec((_GB, NREGION, IN_CH), lambda b: (b, 0, 0)),
            pl.BlockSpec((_GB, NREGION, IN_CH), lambda b: (b, 0, 0)),
            pl.BlockSpec((1, IN_CH), lambda b: (0, 0)),
            pl.BlockSpec((1, IN_CH), lambda b: (0, 0)),
            pl.BlockSpec((1, 2), lambda b: (0, 0)),
        ],
        out_specs=[
            pl.BlockSpec((_GB, npair), lambda b: (b, 0)),
            pl.BlockSpec((_GB, npair), lambda b: (b, 0)),
        ],
        out_shape=[
            jax.ShapeDtypeStruct((ngraph, npair), jnp.float32),
            jax.ShapeDtypeStruct((ngraph, npair), jnp.float32),
        ],
    )(xs3, xd3, W2[0:1, :], W2[1:2, :], b2.reshape(1, 2))

    action = pl.pallas_call(
        _ratio_body,
        grid=(5,),
        in_specs=[
            pl.BlockSpec((ngraph // 5, npair), lambda b: (b, 0)),
            pl.BlockSpec((ngraph // 5, npair), lambda b: (b, 0)),
        ],
        out_specs=pl.BlockSpec((ngraph // 5, npair), lambda b: (b, 0)),
        out_shape=jax.ShapeDtypeStruct((ngraph, npair), jnp.float32),
    )(c0, c1)
    return action.reshape(ngraph, NREGION, NREGION)
